# baseline (speedup 1.0000x reference)
import functools

import jax
import jax.numpy as jnp
from jax import lax
from jax.experimental import pallas as pl
from jax.experimental.pallas import tpu as pltpu
from jax.experimental.pallas import tpu_sc as plsc

EMBED_DIM = 64
BATCH = 16384
NUM_CORES = 2
NUM_SUBCORES = 16
NUM_WORKERS = NUM_CORES * NUM_SUBCORES

B_PER_W = 256
B_SC = NUM_WORKERS * B_PER_W
B_TC = BATCH - B_SC
N_SEMS = 8


def _sc_body(idx_hbm, table_hbm, out_hbm, idx_v, rows_v, sems):
    wid = lax.axis_index("s") * NUM_CORES + lax.axis_index("c")
    pltpu.sync_copy(idx_hbm.at[wid], idx_v)
    copies = []
    for g in range(B_PER_W // 16):
        vec = idx_v[pl.ds(g * 16, 16)]
        for k in range(16):
            j = g * 16 + k
            copies.append(
                pltpu.make_async_copy(
                    table_hbm.at[pl.ds(vec[k], 1)],
                    rows_v.at[pl.ds(j, 1)],
                    sems[j % N_SEMS],
                )
            )
    for cp in copies:
        cp.start()
    for cp in copies:
        cp.wait()
    pltpu.sync_copy(rows_v, out_hbm.at[wid])


@functools.partial(
    pl.kernel,
    out_type=jax.ShapeDtypeStruct(
        (NUM_WORKERS, B_PER_W, EMBED_DIM), jnp.float32
    ),
    mesh=plsc.VectorSubcoreMesh(core_axis_name="c", subcore_axis_name="s"),
    scratch_types=[
        pltpu.VMEM((B_PER_W,), jnp.int32),
        pltpu.VMEM((B_PER_W, EMBED_DIM), jnp.float32),
        [pltpu.SemaphoreType.DMA] * N_SEMS,
    ],
)
def _sc_kernel(idx_hbm, table_hbm, out_hbm, idx_v, rows_v, sems):
    _sc_body(idx_hbm, table_hbm, out_hbm, idx_v, rows_v, sems)


def _tc_body(idx_ref, w_ref, o_ref, sem):
    def issue(j, carry):
        pltpu.make_async_copy(
            w_ref.at[pl.ds(idx_ref[j], 1)], o_ref.at[pl.ds(j, 1)], sem
        ).start()
        return carry

    lax.fori_loop(0, B_TC, issue, 0)
    pltpu.make_async_copy(w_ref.at[pl.ds(0, B_TC)], o_ref, sem).wait()


_tc_kernel = pl.pallas_call(
    _tc_body,
    grid_spec=pltpu.PrefetchScalarGridSpec(
        num_scalar_prefetch=1,
        grid=(1,),
        in_specs=[pl.BlockSpec(memory_space=pltpu.MemorySpace.HBM)],
        out_specs=pl.BlockSpec(memory_space=pltpu.MemorySpace.HBM),
        scratch_shapes=[pltpu.SemaphoreType.DMA],
    ),
    out_shape=jax.ShapeDtypeStruct((B_TC, EMBED_DIM), jnp.float32),
)


def kernel(input_, weight):
    idx = input_.astype(jnp.int32)
    idx_sc = idx[:B_SC].reshape(NUM_WORKERS, B_PER_W)
    idx_tc = idx[B_SC:]
    out_tc = _tc_kernel(idx_tc, weight)
    out_sc = _sc_kernel(idx_sc, weight)
    return jnp.concatenate(
        [out_sc.reshape(B_SC, EMBED_DIM), out_tc], axis=0
    )


# speedup vs baseline: 1.0026x; 1.0026x over previous
import functools

import jax
import jax.numpy as jnp
from jax import lax
from jax.experimental import pallas as pl
from jax.experimental.pallas import tpu as pltpu
from jax.experimental.pallas import tpu_sc as plsc

EMBED_DIM = 64
BATCH = 16384
NUM_CORES = 2
NUM_SUBCORES = 16
NUM_WORKERS = NUM_CORES * NUM_SUBCORES

B_PER_W = 256
B_SC = NUM_WORKERS * B_PER_W
B_TC = BATCH - B_SC
N_SEMS = 8


def _sc_body(idx_hbm, table_hbm, out_hbm, idx_v, rows_v, sems):
    wid = lax.axis_index("s") * NUM_CORES + lax.axis_index("c")
    pltpu.sync_copy(idx_hbm.at[wid], idx_v)
    copies = []
    for g in range(B_PER_W // 16):
        vec = idx_v[pl.ds(g * 16, 16)]
        for k in range(16):
            j = g * 16 + k
            copies.append(
                pltpu.make_async_copy(
                    table_hbm.at[pl.ds(vec[k], 1)],
                    rows_v.at[pl.ds(j, 1)],
                    sems[j % N_SEMS],
                )
            )
    for cp in copies:
        cp.start()
    for cp in copies:
        cp.wait()
    pltpu.sync_copy(rows_v, out_hbm.at[wid])


@functools.partial(
    pl.kernel,
    out_type=jax.ShapeDtypeStruct(
        (NUM_WORKERS, B_PER_W, EMBED_DIM), jnp.float32
    ),
    mesh=plsc.VectorSubcoreMesh(core_axis_name="c", subcore_axis_name="s"),
    scratch_types=[
        pltpu.VMEM((B_PER_W,), jnp.int32),
        pltpu.VMEM((B_PER_W, EMBED_DIM), jnp.float32),
        [pltpu.SemaphoreType.DMA] * N_SEMS,
    ],
    compiler_params=pltpu.CompilerParams(skip_device_barrier=True),
)
def _sc_kernel(idx_hbm, table_hbm, out_hbm, idx_v, rows_v, sems):
    _sc_body(idx_hbm, table_hbm, out_hbm, idx_v, rows_v, sems)


def _tc_body(idx_ref, w_ref, o_ref, sem):
    def issue(j, carry):
        pltpu.make_async_copy(
            w_ref.at[pl.ds(idx_ref[j], 1)], o_ref.at[pl.ds(j, 1)], sem
        ).start()
        return carry

    lax.fori_loop(0, B_TC, issue, 0)
    pltpu.make_async_copy(w_ref.at[pl.ds(0, B_TC)], o_ref, sem).wait()


_tc_kernel = pl.pallas_call(
    _tc_body,
    grid_spec=pltpu.PrefetchScalarGridSpec(
        num_scalar_prefetch=1,
        grid=(1,),
        in_specs=[pl.BlockSpec(memory_space=pltpu.MemorySpace.HBM)],
        out_specs=pl.BlockSpec(memory_space=pltpu.MemorySpace.HBM),
        scratch_shapes=[pltpu.SemaphoreType.DMA],
    ),
    out_shape=jax.ShapeDtypeStruct((B_TC, EMBED_DIM), jnp.float32),
)


def kernel(input_, weight):
    idx = input_.astype(jnp.int32)
    idx_sc = idx[:B_SC].reshape(NUM_WORKERS, B_PER_W)
    idx_tc = idx[B_SC:]
    out_tc = _tc_kernel(idx_tc, weight)
    out_sc = _sc_kernel(idx_sc, weight)
    return jnp.concatenate(
        [out_sc.reshape(B_SC, EMBED_DIM), out_tc], axis=0
    )
